# initial kernel scaffold (unmeasured)
import jax
import jax.numpy as jnp
from jax import lax
from jax.experimental import pallas as pl
from jax.experimental.pallas import tpu as pltpu


def kernel(
    x,
):
    def body(*refs):
        pass

    out_shape = jax.ShapeDtypeStruct(..., jnp.float32)
    return pl.pallas_call(body, out_shape=out_shape)(...)


# baseline (device time: 232898 ns/iter reference)
import jax
import jax.numpy as jnp
from jax import lax
from jax.experimental import pallas as pl
from jax.experimental.pallas import tpu as pltpu

CHUNK = 512


def kernel(x):
    _, m, n2 = x.shape
    n = n2 // 2
    half = m // 2
    nc = half // CHUNK

    def body(x_hbm, out_hbm, recv_ref, loc_ref, ysend, yrecv, xsend, xrecv,
             cp_in, cp_out):
        my_x = lax.axis_index("x")
        my_y = lax.axis_index("y")
        ypeer = (my_x, 1 - my_y)
        xpeer = (1 - my_x, my_y)
        row0 = my_x * half
        prow0 = (1 - my_x) * half

        barrier_sem = pltpu.get_barrier_semaphore()
        for nbr in (ypeer, xpeer):
            pl.semaphore_signal(
                barrier_sem, inc=1, device_id=nbr,
                device_id_type=pl.DeviceIdType.MESH,
            )
        pl.semaphore_wait(barrier_sem, 2)

        y_rdmas = []
        for c in range(nc):
            rdma = pltpu.make_async_remote_copy(
                src_ref=x_hbm.at[0, pl.ds(row0 + c * CHUNK, CHUNK),
                                 pl.ds((1 - my_y) * n, n)],
                dst_ref=recv_ref.at[c],
                send_sem=ysend.at[c],
                recv_sem=yrecv.at[c],
                device_id=ypeer,
                device_id_type=pl.DeviceIdType.MESH,
            )
            rdma.start()
            y_rdmas.append(rdma)

        def start_stage(c):
            cp = pltpu.make_async_copy(
                x_hbm.at[0, pl.ds(row0 + c * CHUNK, CHUNK),
                         pl.ds(my_y * n, n)],
                loc_ref.at[c % 2],
                cp_in.at[c % 2],
            )
            cp.start()
            return cp

        stages = [start_stage(0)]
        x_rdmas = []
        out_cps = []
        for c in range(nc):
            if c + 1 < nc:
                stages.append(start_stage(c + 1))
            y_rdmas[c].wait_recv()
            stages[c].wait()
            recv_ref[c] = recv_ref[c] + loc_ref[c % 2]
            xr = pltpu.make_async_remote_copy(
                src_ref=recv_ref.at[c],
                dst_ref=out_hbm.at[pl.ds(row0 + c * CHUNK, CHUNK), :],
                send_sem=xsend.at[c],
                recv_sem=xrecv.at[c],
                device_id=xpeer,
                device_id_type=pl.DeviceIdType.MESH,
            )
            xr.start()
            x_rdmas.append(xr)
            oc = pltpu.make_async_copy(
                recv_ref.at[c],
                out_hbm.at[pl.ds(row0 + c * CHUNK, CHUNK), :],
                cp_out.at[c],
            )
            oc.start()
            out_cps.append(oc)

        for c in range(nc):
            out_cps[c].wait()
            y_rdmas[c].wait_send()
            x_rdmas[c].wait_send()
            recv_wait = pltpu.make_async_remote_copy(
                src_ref=recv_ref.at[c],
                dst_ref=out_hbm.at[pl.ds(prow0 + c * CHUNK, CHUNK), :],
                send_sem=xsend.at[c],
                recv_sem=xrecv.at[c],
                device_id=xpeer,
                device_id_type=pl.DeviceIdType.MESH,
            )
            recv_wait.wait_recv()

    return pl.pallas_call(
        body,
        out_shape=jax.ShapeDtypeStruct((m, n), jnp.float32),
        in_specs=[pl.BlockSpec(memory_space=pl.ANY)],
        out_specs=pl.BlockSpec(memory_space=pl.ANY),
        scratch_shapes=[
            pltpu.VMEM((half // CHUNK, CHUNK, n), jnp.float32),
            pltpu.VMEM((2, CHUNK, n), jnp.float32),
            pltpu.SemaphoreType.DMA((half // CHUNK,)),
            pltpu.SemaphoreType.DMA((half // CHUNK,)),
            pltpu.SemaphoreType.DMA((half // CHUNK,)),
            pltpu.SemaphoreType.DMA((half // CHUNK,)),
            pltpu.SemaphoreType.DMA((2,)),
            pltpu.SemaphoreType.DMA((half // CHUNK,)),
        ],
        compiler_params=pltpu.CompilerParams(collective_id=0),
    )(x)
